# baseline (speedup 1.0000x reference)
import jax
import jax.numpy as jnp
from jax import lax
from jax.experimental import pallas as pl
from jax.experimental.pallas import tpu as pltpu
from jax.experimental.pallas import tpu_sc as plsc

DIM = 64
NUM_CTX = 5
NC = 2
NS = 16
NW = NC * NS
CB = 128


def _make_body(nchunk):
    def body(tgt_i, ctx_i, tgt_tab, ctx_tab, out,
             tgt_idx_v, ctx_idx_v, tgt_a, tgt_b, ctx_a, ctx_b, out_v,
             sem_a, sem_b):
        w = lax.axis_index("s") * NC + lax.axis_index("c")
        lanes = lax.iota(jnp.int32, 16)
        pltpu.sync_copy(tgt_i.at[w], tgt_idx_v)
        pltpu.sync_copy(ctx_i.at[w], ctx_idx_v)

        tgt_bufs = (tgt_a, tgt_b)
        ctx_bufs = (ctx_a, ctx_b)
        sems = (sem_a, sem_b)

        def fire(k):
            par = k % 2
            waits = [pltpu.async_copy(
                tgt_tab.at[tgt_idx_v.at[k]], tgt_bufs[par], sems[par])]
            for c in range(NUM_CTX):
                waits.append(pltpu.async_copy(
                    ctx_tab.at[ctx_idx_v.at[k, c]],
                    ctx_bufs[par].at[pl.ds(c * CB, CB)], sems[par]))
            return waits

        def unpack2(row_ref, r):
            lo = plsc.unpack(row_ref[r, pl.ds(0, 32)],
                             format=plsc.PackFormat.INTERLEAVED)
            hi = plsc.unpack(row_ref[r, pl.ds(32, 32)],
                             format=plsc.PackFormat.INTERLEAVED)
            return lo + hi

        pending = fire(0)
        for k in range(nchunk):
            for h in pending:
                h.wait()
            if k + 1 < nchunk:
                pending = fire(k + 1)
            tgt_rows = tgt_bufs[k % 2]
            ctx_rows = ctx_bufs[k % 2]
            ks = jnp.full((16,), k, jnp.int32)

            def bstep(b, carry, tgt_rows=tgt_rows, ctx_rows=ctx_rows, ks=ks):
                wv = unpack2(tgt_rows, b)
                vec = jnp.zeros((16,), jnp.float32)
                for s in range(NUM_CTX):
                    xv = unpack2(ctx_rows, b * NUM_CTX + s)
                    acc = wv[0] * xv[0]
                    for i in range(1, 4):
                        acc = acc + wv[i] * xv[i]
                    vec = jnp.where(lanes == s, jnp.sum(acc), vec)
                plsc.store_scatter(out_v, [ks, b * NUM_CTX + lanes], vec,
                                   mask=lanes < NUM_CTX)
                return carry

            lax.fori_loop(0, CB, bstep, 0)

        pltpu.sync_copy(out_v, out.at[w])

    return body


def kernel(target, context, target_table, context_table):
    batch, num_ctx = context.shape
    assert num_ctx == NUM_CTX and batch % (NW * CB) == 0
    nchunk = batch // (NW * CB)
    ppc = CB * NUM_CTX

    tgt_i = target.astype(jnp.int32).reshape(NW, nchunk, CB)
    ctx_i = context.astype(jnp.int32).reshape(NW, nchunk, NUM_CTX, CB)

    mesh = plsc.VectorSubcoreMesh(core_axis_name="c", subcore_axis_name="s")
    grid_kernel = pl.kernel(
        _make_body(nchunk),
        out_type=jax.ShapeDtypeStruct((NW, nchunk, ppc), jnp.float32),
        mesh=mesh,
        scratch_types=[
            pltpu.VMEM((nchunk, CB), jnp.int32),
            pltpu.VMEM((nchunk, NUM_CTX, CB), jnp.int32),
            pltpu.VMEM((CB, DIM), jnp.bfloat16),
            pltpu.VMEM((CB, DIM), jnp.bfloat16),
            pltpu.VMEM((NUM_CTX * CB, DIM), jnp.bfloat16),
            pltpu.VMEM((NUM_CTX * CB, DIM), jnp.bfloat16),
            pltpu.VMEM((nchunk, ppc), jnp.float32),
            pltpu.SemaphoreType.DMA,
            pltpu.SemaphoreType.DMA,
        ],
        compiler_params=pltpu.CompilerParams(
            needs_layout_passes=False, use_tc_tiling_on_sc=False),
    )
    out = grid_kernel(tgt_i, ctx_i,
                      target_table.astype(jnp.bfloat16),
                      context_table.astype(jnp.bfloat16))
    return out.reshape(batch, NUM_CTX)


# speedup vs baseline: 2.1426x; 2.1426x over previous
import jax
import jax.numpy as jnp
from jax import lax
from jax.experimental import pallas as pl
from jax.experimental.pallas import tpu as pltpu
from jax.experimental.pallas import tpu_sc as plsc

DIM = 64
NUM_CTX = 5
NC = 2
NS = 16
NW = NC * NS
CB = 128
VB = 2048
HB = VB // 2


QB = VB // 4
WPR = DIM // 2


def _tc_pack_body(xt_ref, xc_ref, ot_ref, oc_ref):
    for x_ref, o_ref in ((xt_ref, ot_ref), (xc_ref, oc_ref)):
        x = x_ref[...]
        lo = lax.bitcast_convert_type(
            x[0:WPR, :].astype(jnp.bfloat16), jnp.uint16).astype(jnp.uint32)
        hi = lax.bitcast_convert_type(
            x[WPR:DIM, :].astype(jnp.bfloat16), jnp.uint16).astype(jnp.uint32)
        wv = lax.bitcast_convert_type(lo | (hi << 16), jnp.float32)
        wt = jnp.transpose(wv)
        for q in range(4):
            o_ref[:, WPR * q:WPR * (q + 1)] = wt[QB * q:QB * (q + 1), :]


def _tc_pack(tt, ct, grid):
    spec_in = pl.BlockSpec((DIM, VB), lambda i: (0, i))
    spec_out = pl.BlockSpec((QB, 4 * WPR), lambda i: (i, 0))
    out_sds = jax.ShapeDtypeStruct((grid * QB, 4 * WPR), jnp.float32)
    return pl.pallas_call(
        _tc_pack_body,
        grid=(grid,),
        in_specs=[spec_in, spec_in],
        out_specs=[spec_out, spec_out],
        out_shape=[out_sds, out_sds],
        compiler_params=pltpu.CompilerParams(
            dimension_semantics=("arbitrary",)),
    )(tt, ct)


def _make_sc_body(nchunk):
    def body(tgt_i, ctx_i, tgt_tab, ctx_tab, out,
             tgt_idx_v, ctx_idx_v,
             tgt_a, tgt_b, ctx_a, ctx_b, out_v, sem_a, sem_b):
        w = lax.axis_index("s") * NC + lax.axis_index("c")
        lanes = lax.iota(jnp.int32, 16)
        pltpu.sync_copy(tgt_i.at[w], tgt_idx_v)
        pltpu.sync_copy(ctx_i.at[w], ctx_idx_v)

        tgt_bufs = (tgt_a, tgt_b)
        ctx_bufs = (ctx_a, ctx_b)
        sems = (sem_a, sem_b)

        def fire(k):
            par = k % 2
            waits = [pltpu.async_copy(
                tgt_tab.at[tgt_idx_v.at[k]], tgt_bufs[par], sems[par])]
            for c in range(NUM_CTX):
                waits.append(pltpu.async_copy(
                    ctx_tab.at[ctx_idx_v.at[k, c]],
                    ctx_bufs[par].at[pl.ds(c * CB, CB)], sems[par]))
            return waits

        def unpack2(row_ref, r):
            lo = plsc.unpack(plsc.bitcast(row_ref[r, pl.ds(0, 16)],
                                          jnp.bfloat16),
                             format=plsc.PackFormat.INTERLEAVED)
            hi = plsc.unpack(plsc.bitcast(row_ref[r, pl.ds(16, 16)],
                                          jnp.bfloat16),
                             format=plsc.PackFormat.INTERLEAVED)
            return lo + hi

        pending = fire(0)
        for k in range(nchunk):
            for h in pending:
                h.wait()
            if k + 1 < nchunk:
                pending = fire(k + 1)
            tgt_rows = tgt_bufs[k % 2]
            ctx_rows = ctx_bufs[k % 2]
            ks = jnp.full((16,), k, jnp.int32)

            def bstep(b, carry, k=k, tgt_rows=tgt_rows, ctx_rows=ctx_rows,
                      ks=ks):
                wv = unpack2(tgt_rows, b)
                vec = jnp.zeros((16,), jnp.float32)
                for s in range(NUM_CTX):
                    p = b * NUM_CTX + s
                    xv = unpack2(ctx_rows, p)
                    acc = wv[0] * xv[0]
                    for i in range(1, 4):
                        acc = acc + wv[i] * xv[i]
                    vec = jnp.where(lanes == s, jnp.sum(acc), vec)
                plsc.store_scatter(out_v, [ks, b * NUM_CTX + lanes], vec,
                                   mask=lanes < NUM_CTX)
                return carry

            lax.fori_loop(0, CB, bstep, 0)

        pltpu.sync_copy(out_v, out.at[w])

    return body


def kernel(target, context, target_table, context_table):
    batch, num_ctx = context.shape
    vocab = target_table.shape[0]
    assert num_ctx == NUM_CTX and batch % (NW * CB) == 0
    nchunk = batch // (NW * CB)
    ppc = CB * NUM_CTX
    grid = (vocab + VB - 1) // VB

    tpk, cpk = _tc_pack(target_table.T, context_table.T, grid)
    nrows = 4 * grid * QB
    tpk = tpk.reshape(nrows, WPR)
    cpk = cpk.reshape(nrows, WPR)

    def addr(v):
        v = v.astype(jnp.int32)
        return (v >> 11) * (4 * QB) + (v & (QB - 1)) * 4 + ((v >> 9) & 3)

    tgt_i = addr(target).reshape(NW, nchunk, CB)
    ctx_i = addr(context).reshape(NW, nchunk, NUM_CTX, CB)

    mesh = plsc.VectorSubcoreMesh(core_axis_name="c", subcore_axis_name="s")
    grid_kernel = pl.kernel(
        _make_sc_body(nchunk),
        out_type=jax.ShapeDtypeStruct((NW, nchunk, ppc), jnp.float32),
        mesh=mesh,
        scratch_types=[
            pltpu.VMEM((nchunk, CB), jnp.int32),
            pltpu.VMEM((nchunk, NUM_CTX, CB), jnp.int32),
            pltpu.VMEM((CB, WPR), jnp.float32),
            pltpu.VMEM((CB, WPR), jnp.float32),
            pltpu.VMEM((NUM_CTX * CB, WPR), jnp.float32),
            pltpu.VMEM((NUM_CTX * CB, WPR), jnp.float32),
            pltpu.VMEM((nchunk, ppc), jnp.float32),
            pltpu.SemaphoreType.DMA,
            pltpu.SemaphoreType.DMA,
        ],
        compiler_params=pltpu.CompilerParams(
            needs_layout_passes=False, use_tc_tiling_on_sc=False),
    )
    out = grid_kernel(tgt_i, ctx_i, tpk, cpk)
    return out.reshape(batch, NUM_CTX)
